# Initial kernel scaffold; baseline (speedup 1.0000x reference)
import jax
import jax.numpy as jnp
from jax.experimental import pallas as pl


def kernel(x, edge_index, W1, b1, W2, b2, Wc, bc):
    raise NotImplementedError("write your pallas kernel here")


# speedup vs baseline: 12.2343x; 12.2343x over previous
import functools

import jax
import jax.numpy as jnp
from jax import lax
from jax.experimental import pallas as pl
from jax.experimental.pallas import tpu as pltpu
from jax.experimental.pallas import tpu_sc as plsc

N_NODES = 10000
NPAD = 10240
D = 128
NC, NS = 2, 16
NW = NC * NS
K = 128
EPW = 10240
EPAD = EPW * NW
CHUNKS = EPW // K
ZPT = NPAD // NS

_mesh = plsc.VectorSubcoreMesh(core_axis_name="c", subcore_axis_name="s")


@functools.partial(
    pl.kernel,
    out_type=jax.ShapeDtypeStruct((NC * NPAD, D), jnp.float32),
    mesh=_mesh,
    scratch_types=[
        pltpu.VMEM((K,), jnp.int32),
        pltpu.VMEM((K,), jnp.int32),
        pltpu.VMEM((K, D), jnp.float32),
        pltpu.VMEM_SHARED((NPAD, D), jnp.float32),
        pltpu.SemaphoreType.DMA,
    ],
)
def _agg_partials(src_hbm, dst_hbm, h_hbm, out_hbm, src_v, dst_v, rows_v, acc_sh, sem):
    c = lax.axis_index("c")
    s = lax.axis_index("s")
    w = c * NS + s

    def zero(i, carry):
        for j in range(D // 16):
            rows_v[i, pl.ds(j * 16, 16)] = jnp.zeros((16,), jnp.float32)
        return carry

    lax.fori_loop(0, K, zero, 0)
    for j in range(ZPT // K):
        pltpu.sync_copy(rows_v, acc_sh.at[pl.ds(s * ZPT + j * K, K)])
    plsc.subcore_barrier()

    def chunk(t, carry):
        off = w * EPW + t * K
        pltpu.sync_copy(src_hbm.at[pl.ds(off, K)], src_v)
        pltpu.sync_copy(dst_hbm.at[pl.ds(off, K)], dst_v)
        pltpu.async_copy(h_hbm.at[src_v], rows_v, sem).wait()
        pltpu.sync_copy(rows_v, acc_sh.at[dst_v], add=True)
        return carry

    lax.fori_loop(0, CHUNKS, chunk, 0)
    plsc.subcore_barrier()
    pltpu.sync_copy(
        acc_sh.at[pl.ds(s * ZPT, ZPT)], out_hbm.at[pl.ds(c * NPAD + s * ZPT, ZPT)]
    )


def _agg(src, dst, h):
    return _agg_partials(src, dst, h).reshape(NC, NPAD, D)


BLK = 1024


def _dinv_block(degp):
    deg = 1.0 + degp[0] + degp[1]
    return lax.rsqrt(deg)


def _tc1_body(x_ref, w_ref, degp_ref, o_ref):
    dinv = _dinv_block(degp_ref[...])
    h = jnp.dot(x_ref[...], w_ref[...], preferred_element_type=jnp.float32)
    o_ref[...] = h * dinv


_tc1 = pl.pallas_call(
    _tc1_body,
    grid=(NPAD // BLK,),
    in_specs=[
        pl.BlockSpec((BLK, D), lambda i: (i, 0)),
        pl.BlockSpec((D, D), lambda i: (0, 0)),
        pl.BlockSpec((NC, BLK, 1), lambda i: (0, i, 0)),
    ],
    out_specs=pl.BlockSpec((BLK, D), lambda i: (i, 0)),
    out_shape=jax.ShapeDtypeStruct((NPAD, D), jnp.float32),
)


def _tc2_body(p_ref, hp_ref, degp_ref, b_ref, w_ref, o_ref):
    dinv = _dinv_block(degp_ref[...])
    ssum = p_ref[0] + p_ref[1] + hp_ref[...]
    a = jnp.maximum(ssum * dinv + b_ref[...], 0.0)
    h = jnp.dot(a, w_ref[...], preferred_element_type=jnp.float32)
    o_ref[...] = h * dinv


_tc2 = pl.pallas_call(
    _tc2_body,
    grid=(NPAD // BLK,),
    in_specs=[
        pl.BlockSpec((NC, BLK, D), lambda i: (0, i, 0)),
        pl.BlockSpec((BLK, D), lambda i: (i, 0)),
        pl.BlockSpec((NC, BLK, 1), lambda i: (0, i, 0)),
        pl.BlockSpec((1, D), lambda i: (0, 0)),
        pl.BlockSpec((D, D), lambda i: (0, 0)),
    ],
    out_specs=pl.BlockSpec((BLK, D), lambda i: (i, 0)),
    out_shape=jax.ShapeDtypeStruct((NPAD, D), jnp.float32),
)


def _tc3_body(p_ref, hp_ref, degp_ref, b_ref, wc_ref, bc_ref, o_ref):
    dinv = _dinv_block(degp_ref[...])
    ssum = p_ref[0] + p_ref[1] + hp_ref[...]
    a = jnp.maximum(ssum * dinv + b_ref[...], 0.0)
    logits = jnp.dot(a, wc_ref[...], preferred_element_type=jnp.float32) + bc_ref[...]
    m = jnp.max(logits, axis=1, keepdims=True)
    lse = jnp.log(jnp.sum(jnp.exp(logits - m), axis=1, keepdims=True))
    o_ref[...] = logits - m - lse


def _make_tc3(n_classes):
    return pl.pallas_call(
        _tc3_body,
        grid=(NPAD // BLK,),
        in_specs=[
            pl.BlockSpec((NC, BLK, D), lambda i: (0, i, 0)),
            pl.BlockSpec((BLK, D), lambda i: (i, 0)),
            pl.BlockSpec((NC, BLK, 1), lambda i: (0, i, 0)),
            pl.BlockSpec((1, D), lambda i: (0, 0)),
            pl.BlockSpec((D, n_classes), lambda i: (0, 0)),
            pl.BlockSpec((1, n_classes), lambda i: (0, 0)),
        ],
        out_specs=pl.BlockSpec((BLK, n_classes), lambda i: (i, 0)),
        out_shape=jax.ShapeDtypeStruct((NPAD, n_classes), jnp.float32),
    )


def kernel(x, edge_index, W1, b1, W2, b2, Wc, bc):
    n_classes = Wc.shape[1]
    n = x.shape[0]
    e = edge_index.shape[1]
    pad = EPAD - e
    pad_ids = jnp.arange(pad, dtype=jnp.int32)
    src = jnp.concatenate([edge_index[0].astype(jnp.int32), pad_ids % n])
    dst = jnp.concatenate(
        [edge_index[1].astype(jnp.int32), n + pad_ids % (NPAD - n)]
    )
    xp = jnp.concatenate([x, jnp.zeros((NPAD - n, D), x.dtype)])

    degp = _agg(src, dst, jnp.ones((NPAD, D), jnp.float32))[:, :, 0:1]
    h1p = _tc1(xp, W1, degp)
    p1 = _agg(src, dst, h1p)
    h2p = _tc2(p1, h1p, degp, b1.reshape(1, D), W2)
    p2 = _agg(src, dst, h2p)
    out = _make_tc3(n_classes)(
        p2, h2p, degp, b2.reshape(1, D), Wc, bc.reshape(1, n_classes)
    )
    return out[:n]


# speedup vs baseline: 16.7142x; 1.3662x over previous
import functools

import jax
import jax.numpy as jnp
from jax import lax
from jax.experimental import pallas as pl
from jax.experimental.pallas import tpu as pltpu
from jax.experimental.pallas import tpu_sc as plsc

N_NODES = 10000
NPAD = 10240
D = 128
NC, NS = 2, 16
NW = NC * NS
K = 128
EPW = 10240
EPAD = EPW * NW
CHUNKS = EPW // K
ZPT = NPAD // NS

UNROLL = 2
OUTER = CHUNKS // UNROLL

_mesh = plsc.VectorSubcoreMesh(core_axis_name="c", subcore_axis_name="s")


@functools.partial(
    pl.kernel,
    out_type=jax.ShapeDtypeStruct((NC * NPAD, D), jnp.float32),
    mesh=_mesh,
    scratch_types=[
        [pltpu.VMEM((K,), jnp.int32)] * UNROLL,
        [pltpu.VMEM((K,), jnp.int32)] * UNROLL,
        pltpu.VMEM((UNROLL, K, D), jnp.float32),
        pltpu.VMEM_SHARED((NPAD, D), jnp.float32),
        [pltpu.SemaphoreType.DMA] * UNROLL,
        [pltpu.SemaphoreType.DMA] * UNROLL,
    ],
)
def _agg_partials(src_hbm, dst_hbm, h_hbm, out_hbm, si_v, di_v, rows_v, acc_sh,
                  gsems, ssems):
    c = lax.axis_index("c")
    s = lax.axis_index("s")
    w = c * NS + s

    def zero(i, carry):
        for j in range(D // 16):
            rows_v[0, i, pl.ds(j * 16, 16)] = jnp.zeros((16,), jnp.float32)
        return carry

    lax.fori_loop(0, K, zero, 0)
    for j in range(ZPT // K):
        pltpu.sync_copy(rows_v.at[0], acc_sh.at[pl.ds(s * ZPT + j * K, K)])
    plsc.subcore_barrier()

    def outer(t, carry):
        base = w * EPW + t * UNROLL * K
        gds = []
        for b in range(UNROLL):
            pltpu.sync_copy(src_hbm.at[pl.ds(base + b * K, K)], si_v[b])
            pltpu.sync_copy(dst_hbm.at[pl.ds(base + b * K, K)], di_v[b])
            gds.append(
                pltpu.async_copy(h_hbm.at[si_v[b]], rows_v.at[b], gsems[b])
            )
        sds = []
        for b in range(UNROLL):
            gds[b].wait()
            sds.append(
                pltpu.async_copy(
                    rows_v.at[b], acc_sh.at[di_v[b]], ssems[b], add=True
                )
            )
        for b in range(UNROLL):
            sds[b].wait()
        return carry

    lax.fori_loop(0, OUTER, outer, 0)
    plsc.subcore_barrier()
    pltpu.sync_copy(
        acc_sh.at[pl.ds(s * ZPT, ZPT)], out_hbm.at[pl.ds(c * NPAD + s * ZPT, ZPT)]
    )


def _agg(src, dst, h):
    return _agg_partials(src, dst, h).reshape(NC, NPAD, D)


BLK = 1024


def _dinv_block(degp):
    deg = 1.0 + degp[0] + degp[1]
    return lax.rsqrt(deg)


def _tc1_body(x_ref, w_ref, degp_ref, o_ref):
    dinv = _dinv_block(degp_ref[...])
    h = jnp.dot(x_ref[...], w_ref[...], preferred_element_type=jnp.float32)
    o_ref[...] = h * dinv


_tc1 = pl.pallas_call(
    _tc1_body,
    grid=(NPAD // BLK,),
    in_specs=[
        pl.BlockSpec((BLK, D), lambda i: (i, 0)),
        pl.BlockSpec((D, D), lambda i: (0, 0)),
        pl.BlockSpec((NC, BLK, 1), lambda i: (0, i, 0)),
    ],
    out_specs=pl.BlockSpec((BLK, D), lambda i: (i, 0)),
    out_shape=jax.ShapeDtypeStruct((NPAD, D), jnp.float32),
)


def _tc2_body(p_ref, hp_ref, degp_ref, b_ref, w_ref, o_ref):
    dinv = _dinv_block(degp_ref[...])
    ssum = p_ref[0] + p_ref[1] + hp_ref[...]
    a = jnp.maximum(ssum * dinv + b_ref[...], 0.0)
    h = jnp.dot(a, w_ref[...], preferred_element_type=jnp.float32)
    o_ref[...] = h * dinv


_tc2 = pl.pallas_call(
    _tc2_body,
    grid=(NPAD // BLK,),
    in_specs=[
        pl.BlockSpec((NC, BLK, D), lambda i: (0, i, 0)),
        pl.BlockSpec((BLK, D), lambda i: (i, 0)),
        pl.BlockSpec((NC, BLK, 1), lambda i: (0, i, 0)),
        pl.BlockSpec((1, D), lambda i: (0, 0)),
        pl.BlockSpec((D, D), lambda i: (0, 0)),
    ],
    out_specs=pl.BlockSpec((BLK, D), lambda i: (i, 0)),
    out_shape=jax.ShapeDtypeStruct((NPAD, D), jnp.float32),
)


def _tc3_body(p_ref, hp_ref, degp_ref, b_ref, wc_ref, bc_ref, o_ref):
    dinv = _dinv_block(degp_ref[...])
    ssum = p_ref[0] + p_ref[1] + hp_ref[...]
    a = jnp.maximum(ssum * dinv + b_ref[...], 0.0)
    logits = jnp.dot(a, wc_ref[...], preferred_element_type=jnp.float32) + bc_ref[...]
    m = jnp.max(logits, axis=1, keepdims=True)
    lse = jnp.log(jnp.sum(jnp.exp(logits - m), axis=1, keepdims=True))
    o_ref[...] = logits - m - lse


def _make_tc3(n_classes):
    return pl.pallas_call(
        _tc3_body,
        grid=(NPAD // BLK,),
        in_specs=[
            pl.BlockSpec((NC, BLK, D), lambda i: (0, i, 0)),
            pl.BlockSpec((BLK, D), lambda i: (i, 0)),
            pl.BlockSpec((NC, BLK, 1), lambda i: (0, i, 0)),
            pl.BlockSpec((1, D), lambda i: (0, 0)),
            pl.BlockSpec((D, n_classes), lambda i: (0, 0)),
            pl.BlockSpec((1, n_classes), lambda i: (0, 0)),
        ],
        out_specs=pl.BlockSpec((BLK, n_classes), lambda i: (i, 0)),
        out_shape=jax.ShapeDtypeStruct((NPAD, n_classes), jnp.float32),
    )


def kernel(x, edge_index, W1, b1, W2, b2, Wc, bc):
    n_classes = Wc.shape[1]
    n = x.shape[0]
    e = edge_index.shape[1]
    pad = EPAD - e
    pad_ids = jnp.arange(pad, dtype=jnp.int32)
    src = jnp.concatenate([edge_index[0].astype(jnp.int32), pad_ids % n])
    dst = jnp.concatenate(
        [edge_index[1].astype(jnp.int32), n + pad_ids % (NPAD - n)]
    )
    xp = jnp.concatenate([x, jnp.zeros((NPAD - n, D), x.dtype)])

    degp = _agg(src, dst, jnp.ones((NPAD, D), jnp.float32))[:, :, 0:1]
    h1p = _tc1(xp, W1, degp)
    p1 = _agg(src, dst, h1p)
    h2p = _tc2(p1, h1p, degp, b1.reshape(1, D), W2)
    p2 = _agg(src, dst, h2p)
    out = _make_tc3(n_classes)(
        p2, h2p, degp, b2.reshape(1, D), Wc, bc.reshape(1, n_classes)
    )
    return out[:n]


# speedup vs baseline: 18.5597x; 1.1104x over previous
import functools

import jax
import jax.numpy as jnp
from jax import lax
from jax.experimental import pallas as pl
from jax.experimental.pallas import tpu as pltpu
from jax.experimental.pallas import tpu_sc as plsc

N_NODES = 10000
NPAD = 10240
D = 128
NC, NS = 2, 16
NW = NC * NS
K = 80
EPW = 10240
EPAD = EPW * NW
CHUNKS = EPW // K
ZPT = NPAD // NS

UNROLL = 4
OUTER = CHUNKS // UNROLL

_mesh = plsc.VectorSubcoreMesh(core_axis_name="c", subcore_axis_name="s")


@functools.partial(
    pl.kernel,
    out_type=jax.ShapeDtypeStruct((NC * NPAD, D), jnp.float32),
    mesh=_mesh,
    scratch_types=[
        [pltpu.VMEM((K,), jnp.int32)] * UNROLL,
        [pltpu.VMEM((K,), jnp.int32)] * UNROLL,
        pltpu.VMEM((UNROLL, K, D), jnp.float32),
        pltpu.VMEM_SHARED((NPAD, D), jnp.float32),
        [pltpu.SemaphoreType.DMA] * UNROLL,
        [pltpu.SemaphoreType.DMA] * UNROLL,
        [pltpu.SemaphoreType.DMA] * UNROLL,
        [pltpu.SemaphoreType.DMA] * UNROLL,
    ],
)
def _agg_partials(src_hbm, dst_hbm, h_hbm, out_hbm, si_v, di_v, rows_v, acc_sh,
                  gsems, ssems, isems, dsems):
    c = lax.axis_index("c")
    s = lax.axis_index("s")
    w = c * NS + s

    def zero(i, carry):
        for j in range(D // 16):
            rows_v[0, i, pl.ds(j * 16, 16)] = jnp.zeros((16,), jnp.float32)
        return carry

    lax.fori_loop(0, K, zero, 0)
    for j in range(ZPT // K):
        pltpu.sync_copy(rows_v.at[0], acc_sh.at[pl.ds(s * ZPT + j * K, K)])
    plsc.subcore_barrier()

    def outer(t, carry):
        base = w * EPW + t * UNROLL * K
        ids, dds = [], []
        for b in range(UNROLL):
            ids.append(
                pltpu.async_copy(src_hbm.at[pl.ds(base + b * K, K)], si_v[b], isems[b])
            )
            dds.append(
                pltpu.async_copy(dst_hbm.at[pl.ds(base + b * K, K)], di_v[b], dsems[b])
            )
        gds = []
        for b in range(UNROLL):
            ids[b].wait()
            gds.append(
                pltpu.async_copy(h_hbm.at[si_v[b]], rows_v.at[b], gsems[b])
            )
        sds = []
        for b in range(UNROLL):
            gds[b].wait()
            dds[b].wait()
            sds.append(
                pltpu.async_copy(
                    rows_v.at[b], acc_sh.at[di_v[b]], ssems[b], add=True
                )
            )
        for b in range(UNROLL):
            sds[b].wait()
        return carry

    lax.fori_loop(0, OUTER, outer, 0)
    plsc.subcore_barrier()
    pltpu.sync_copy(
        acc_sh.at[pl.ds(s * ZPT, ZPT)], out_hbm.at[pl.ds(c * NPAD + s * ZPT, ZPT)]
    )


def _agg(src, dst, h):
    return _agg_partials(src, dst, h).reshape(NC, NPAD, D)


BLK = 1024


def _dinv_block(degp):
    deg = 1.0 + degp[0] + degp[1]
    return lax.rsqrt(deg)


def _tc1_body(x_ref, w_ref, degp_ref, o_ref):
    dinv = _dinv_block(degp_ref[...])
    h = jnp.dot(x_ref[...], w_ref[...], preferred_element_type=jnp.float32)
    o_ref[...] = h * dinv


_tc1 = pl.pallas_call(
    _tc1_body,
    grid=(NPAD // BLK,),
    in_specs=[
        pl.BlockSpec((BLK, D), lambda i: (i, 0)),
        pl.BlockSpec((D, D), lambda i: (0, 0)),
        pl.BlockSpec((NC, BLK, 1), lambda i: (0, i, 0)),
    ],
    out_specs=pl.BlockSpec((BLK, D), lambda i: (i, 0)),
    out_shape=jax.ShapeDtypeStruct((NPAD, D), jnp.float32),
)


def _tc2_body(p_ref, hp_ref, degp_ref, b_ref, w_ref, o_ref):
    dinv = _dinv_block(degp_ref[...])
    ssum = p_ref[0] + p_ref[1] + hp_ref[...]
    a = jnp.maximum(ssum * dinv + b_ref[...], 0.0)
    h = jnp.dot(a, w_ref[...], preferred_element_type=jnp.float32)
    o_ref[...] = h * dinv


_tc2 = pl.pallas_call(
    _tc2_body,
    grid=(NPAD // BLK,),
    in_specs=[
        pl.BlockSpec((NC, BLK, D), lambda i: (0, i, 0)),
        pl.BlockSpec((BLK, D), lambda i: (i, 0)),
        pl.BlockSpec((NC, BLK, 1), lambda i: (0, i, 0)),
        pl.BlockSpec((1, D), lambda i: (0, 0)),
        pl.BlockSpec((D, D), lambda i: (0, 0)),
    ],
    out_specs=pl.BlockSpec((BLK, D), lambda i: (i, 0)),
    out_shape=jax.ShapeDtypeStruct((NPAD, D), jnp.float32),
)


def _tc3_body(p_ref, hp_ref, degp_ref, b_ref, wc_ref, bc_ref, o_ref):
    dinv = _dinv_block(degp_ref[...])
    ssum = p_ref[0] + p_ref[1] + hp_ref[...]
    a = jnp.maximum(ssum * dinv + b_ref[...], 0.0)
    logits = jnp.dot(a, wc_ref[...], preferred_element_type=jnp.float32) + bc_ref[...]
    m = jnp.max(logits, axis=1, keepdims=True)
    lse = jnp.log(jnp.sum(jnp.exp(logits - m), axis=1, keepdims=True))
    o_ref[...] = logits - m - lse


def _make_tc3(n_classes):
    return pl.pallas_call(
        _tc3_body,
        grid=(NPAD // BLK,),
        in_specs=[
            pl.BlockSpec((NC, BLK, D), lambda i: (0, i, 0)),
            pl.BlockSpec((BLK, D), lambda i: (i, 0)),
            pl.BlockSpec((NC, BLK, 1), lambda i: (0, i, 0)),
            pl.BlockSpec((1, D), lambda i: (0, 0)),
            pl.BlockSpec((D, n_classes), lambda i: (0, 0)),
            pl.BlockSpec((1, n_classes), lambda i: (0, 0)),
        ],
        out_specs=pl.BlockSpec((BLK, n_classes), lambda i: (i, 0)),
        out_shape=jax.ShapeDtypeStruct((NPAD, n_classes), jnp.float32),
    )


def kernel(x, edge_index, W1, b1, W2, b2, Wc, bc):
    n_classes = Wc.shape[1]
    n = x.shape[0]
    e = edge_index.shape[1]
    pad = EPAD - e
    pad_ids = jnp.arange(pad, dtype=jnp.int32)
    src = jnp.concatenate([edge_index[0].astype(jnp.int32), pad_ids % n])
    dst = jnp.concatenate(
        [edge_index[1].astype(jnp.int32), n + pad_ids % (NPAD - n)]
    )
    xp = jnp.concatenate([x, jnp.zeros((NPAD - n, D), x.dtype)])

    degp = _agg(src, dst, jnp.ones((NPAD, D), jnp.float32))[:, :, 0:1]
    h1p = _tc1(xp, W1, degp)
    p1 = _agg(src, dst, h1p)
    h2p = _tc2(p1, h1p, degp, b1.reshape(1, D), W2)
    p2 = _agg(src, dst, h2p)
    out = _make_tc3(n_classes)(
        p2, h2p, degp, b2.reshape(1, D), Wc, bc.reshape(1, n_classes)
    )
    return out[:n]


# speedup vs baseline: 21.6783x; 1.1680x over previous
import functools

import jax
import jax.numpy as jnp
from jax import lax
from jax.experimental import pallas as pl
from jax.experimental.pallas import tpu as pltpu
from jax.experimental.pallas import tpu_sc as plsc

N_NODES = 10000
NPAD = 10240
D = 128
NC, NS = 2, 16
NW = NC * NS
K = 80
EPW = 10240
EPAD = EPW * NW
CHUNKS = EPW // K
ZPT = NPAD // NS

UNROLL = 4
OUTER = CHUNKS // UNROLL

_mesh = plsc.VectorSubcoreMesh(core_axis_name="c", subcore_axis_name="s")


@functools.partial(
    pl.kernel,
    out_type=jax.ShapeDtypeStruct((NC * NPAD, D), jnp.float32),
    mesh=_mesh,
    scratch_types=[
        [pltpu.VMEM((K,), jnp.int32)] * UNROLL,
        [pltpu.VMEM((K,), jnp.int32)] * UNROLL,
        pltpu.VMEM((UNROLL, K, D), jnp.float32),
        pltpu.VMEM_SHARED((NPAD, D), jnp.float32),
        [pltpu.SemaphoreType.DMA] * UNROLL,
        [pltpu.SemaphoreType.DMA] * UNROLL,
        [pltpu.SemaphoreType.DMA] * UNROLL,
        [pltpu.SemaphoreType.DMA] * UNROLL,
    ],
)
def _agg_partials(src_hbm, dst_hbm, h_hbm, out_hbm, si_v, di_v, rows_v, acc_sh,
                  gsems, ssems, isems, dsems):
    c = lax.axis_index("c")
    s = lax.axis_index("s")
    w = c * NS + s

    def zero(i, carry):
        for j in range(D // 16):
            rows_v[0, i, pl.ds(j * 16, 16)] = jnp.zeros((16,), jnp.float32)
        return carry

    lax.fori_loop(0, K, zero, 0)
    for j in range(ZPT // K):
        pltpu.sync_copy(rows_v.at[0], acc_sh.at[pl.ds(s * ZPT + j * K, K)])
    plsc.subcore_barrier()

    def outer(t, carry):
        base = w * EPW + t * UNROLL * K
        ids, dds = [], []
        for b in range(UNROLL):
            ids.append(
                pltpu.async_copy(src_hbm.at[pl.ds(base + b * K, K)], si_v[b], isems[b])
            )
            dds.append(
                pltpu.async_copy(dst_hbm.at[pl.ds(base + b * K, K)], di_v[b], dsems[b])
            )
        gds = []
        for b in range(UNROLL):
            ids[b].wait()
            gds.append(
                pltpu.async_copy(h_hbm.at[si_v[b]], rows_v.at[b], gsems[b])
            )
        sds = []
        for b in range(UNROLL):
            gds[b].wait()
            dds[b].wait()
            sds.append(
                pltpu.async_copy(
                    rows_v.at[b], acc_sh.at[di_v[b]], ssems[b], add=True
                )
            )
        for b in range(UNROLL):
            sds[b].wait()
        return carry

    lax.fori_loop(0, OUTER, outer, 0)
    plsc.subcore_barrier()
    pltpu.sync_copy(
        acc_sh.at[pl.ds(s * ZPT, ZPT)], out_hbm.at[pl.ds(c * NPAD + s * ZPT, ZPT)]
    )


@functools.partial(
    pl.kernel,
    out_type=jax.ShapeDtypeStruct((NC * NPAD, D), jnp.float32),
    mesh=_mesh,
    scratch_types=[
        [pltpu.VMEM((K,), jnp.int32)] * UNROLL,
        pltpu.VMEM((K, D), jnp.float32),
        pltpu.VMEM_SHARED((NPAD, D), jnp.float32),
        [pltpu.SemaphoreType.DMA] * UNROLL,
        [pltpu.SemaphoreType.DMA] * UNROLL,
    ],
)
def _deg_partials(dst_hbm, out_hbm, di_v, ones_v, acc_sh, ssems, dsems):
    c = lax.axis_index("c")
    s = lax.axis_index("s")
    w = c * NS + s

    def zero(i, carry):
        for j in range(D // 16):
            ones_v[i, pl.ds(j * 16, 16)] = jnp.zeros((16,), jnp.float32)
        return carry

    lax.fori_loop(0, K, zero, 0)
    for j in range(ZPT // K):
        pltpu.sync_copy(ones_v, acc_sh.at[pl.ds(s * ZPT + j * K, K)])

    def fill(i, carry):
        for j in range(D // 16):
            ones_v[i, pl.ds(j * 16, 16)] = jnp.full((16,), 1.0, jnp.float32)
        return carry

    lax.fori_loop(0, K, fill, 0)
    plsc.subcore_barrier()

    def outer(t, carry):
        base = w * EPW + t * UNROLL * K
        dds = [
            pltpu.async_copy(dst_hbm.at[pl.ds(base + b * K, K)], di_v[b], dsems[b])
            for b in range(UNROLL)
        ]
        sds = []
        for b in range(UNROLL):
            dds[b].wait()
            sds.append(
                pltpu.async_copy(ones_v, acc_sh.at[di_v[b]], ssems[b], add=True)
            )
        for b in range(UNROLL):
            sds[b].wait()
        return carry

    lax.fori_loop(0, OUTER, outer, 0)
    plsc.subcore_barrier()
    pltpu.sync_copy(
        acc_sh.at[pl.ds(s * ZPT, ZPT)], out_hbm.at[pl.ds(c * NPAD + s * ZPT, ZPT)]
    )


def _agg(src, dst, h):
    return _agg_partials(src, dst, h).reshape(NC, NPAD, D)


BLK = 1024


def _dinv_block(degp):
    deg = 1.0 + degp[0] + degp[1]
    return lax.rsqrt(deg)


def _tc1_body(x_ref, w_ref, degp_ref, o_ref):
    dinv = _dinv_block(degp_ref[...])
    h = jnp.dot(x_ref[...], w_ref[...], preferred_element_type=jnp.float32)
    o_ref[...] = h * dinv


_tc1 = pl.pallas_call(
    _tc1_body,
    grid=(NPAD // BLK,),
    in_specs=[
        pl.BlockSpec((BLK, D), lambda i: (i, 0)),
        pl.BlockSpec((D, D), lambda i: (0, 0)),
        pl.BlockSpec((NC, BLK, 1), lambda i: (0, i, 0)),
    ],
    out_specs=pl.BlockSpec((BLK, D), lambda i: (i, 0)),
    out_shape=jax.ShapeDtypeStruct((NPAD, D), jnp.float32),
)


def _tc2_body(p_ref, hp_ref, degp_ref, b_ref, w_ref, o_ref):
    dinv = _dinv_block(degp_ref[...])
    ssum = p_ref[0] + p_ref[1] + hp_ref[...]
    a = jnp.maximum(ssum * dinv + b_ref[...], 0.0)
    h = jnp.dot(a, w_ref[...], preferred_element_type=jnp.float32)
    o_ref[...] = h * dinv


_tc2 = pl.pallas_call(
    _tc2_body,
    grid=(NPAD // BLK,),
    in_specs=[
        pl.BlockSpec((NC, BLK, D), lambda i: (0, i, 0)),
        pl.BlockSpec((BLK, D), lambda i: (i, 0)),
        pl.BlockSpec((NC, BLK, 1), lambda i: (0, i, 0)),
        pl.BlockSpec((1, D), lambda i: (0, 0)),
        pl.BlockSpec((D, D), lambda i: (0, 0)),
    ],
    out_specs=pl.BlockSpec((BLK, D), lambda i: (i, 0)),
    out_shape=jax.ShapeDtypeStruct((NPAD, D), jnp.float32),
)


def _tc3_body(p_ref, hp_ref, degp_ref, b_ref, wc_ref, bc_ref, o_ref):
    dinv = _dinv_block(degp_ref[...])
    ssum = p_ref[0] + p_ref[1] + hp_ref[...]
    a = jnp.maximum(ssum * dinv + b_ref[...], 0.0)
    logits = jnp.dot(a, wc_ref[...], preferred_element_type=jnp.float32) + bc_ref[...]
    m = jnp.max(logits, axis=1, keepdims=True)
    lse = jnp.log(jnp.sum(jnp.exp(logits - m), axis=1, keepdims=True))
    o_ref[...] = logits - m - lse


def _make_tc3(n_classes):
    return pl.pallas_call(
        _tc3_body,
        grid=(NPAD // BLK,),
        in_specs=[
            pl.BlockSpec((NC, BLK, D), lambda i: (0, i, 0)),
            pl.BlockSpec((BLK, D), lambda i: (i, 0)),
            pl.BlockSpec((NC, BLK, 1), lambda i: (0, i, 0)),
            pl.BlockSpec((1, D), lambda i: (0, 0)),
            pl.BlockSpec((D, n_classes), lambda i: (0, 0)),
            pl.BlockSpec((1, n_classes), lambda i: (0, 0)),
        ],
        out_specs=pl.BlockSpec((BLK, n_classes), lambda i: (i, 0)),
        out_shape=jax.ShapeDtypeStruct((NPAD, n_classes), jnp.float32),
    )


def kernel(x, edge_index, W1, b1, W2, b2, Wc, bc):
    n_classes = Wc.shape[1]
    n = x.shape[0]
    e = edge_index.shape[1]
    pad = EPAD - e
    pad_ids = jnp.arange(pad, dtype=jnp.int32)
    src = jnp.concatenate([edge_index[0].astype(jnp.int32), pad_ids % n])
    dst = jnp.concatenate(
        [edge_index[1].astype(jnp.int32), n + pad_ids % (NPAD - n)]
    )
    xp = jnp.concatenate([x, jnp.zeros((NPAD - n, D), x.dtype)])

    degp = _deg_partials(dst).reshape(NC, NPAD, D)[:, :, 0:1]
    h1p = _tc1(xp, W1, degp)
    p1 = _agg(src, dst, h1p)
    h2p = _tc2(p1, h1p, degp, b1.reshape(1, D), W2)
    p2 = _agg(src, dst, h2p)
    out = _make_tc3(n_classes)(
        p2, h2p, degp, b2.reshape(1, D), Wc, bc.reshape(1, n_classes)
    )
    return out[:n]


# speedup vs baseline: 24.1320x; 1.1132x over previous
import functools

import jax
import jax.numpy as jnp
from jax import lax
from jax.experimental import pallas as pl
from jax.experimental.pallas import tpu as pltpu
from jax.experimental.pallas import tpu_sc as plsc

N_NODES = 10000
NPAD = 10240
D = 128
NC, NS = 2, 16
NW = NC * NS
K = 80
EPW = 10240
EPAD = EPW * NW
CHUNKS = EPW // K
ZPT = NPAD // NS

UNROLL = 4
OUTER = CHUNKS // UNROLL

_mesh = plsc.VectorSubcoreMesh(core_axis_name="c", subcore_axis_name="s")


@functools.partial(
    pl.kernel,
    out_type=jax.ShapeDtypeStruct((NC * NPAD, D), jnp.float32),
    mesh=_mesh,
    scratch_types=[
        [pltpu.VMEM((K,), jnp.int32)] * UNROLL,
        [pltpu.VMEM((K,), jnp.int32)] * UNROLL,
        pltpu.VMEM((UNROLL, K, D), jnp.float32),
        pltpu.VMEM_SHARED((NPAD, D), jnp.float32),
        [pltpu.SemaphoreType.DMA] * UNROLL,
        [pltpu.SemaphoreType.DMA] * UNROLL,
        [pltpu.SemaphoreType.DMA] * UNROLL,
        [pltpu.SemaphoreType.DMA] * UNROLL,
    ],
)
def _agg_partials(src_hbm, dst_hbm, h_hbm, out_hbm, si_v, di_v, rows_v, acc_sh,
                  gsems, ssems, isems, dsems):
    c = lax.axis_index("c")
    s = lax.axis_index("s")
    w = c * NS + s

    def zero(i, carry):
        for j in range(D // 16):
            rows_v[0, i, pl.ds(j * 16, 16)] = jnp.zeros((16,), jnp.float32)
        return carry

    lax.fori_loop(0, K, zero, 0)
    for j in range(ZPT // K):
        pltpu.sync_copy(rows_v.at[0], acc_sh.at[pl.ds(s * ZPT + j * K, K)])
    plsc.subcore_barrier()

    def body(t, drain):
        base = w * EPW + t * UNROLL * K
        ids, dds = [], []
        for b in range(UNROLL):
            if drain:
                pltpu.make_async_copy(
                    rows_v.at[b], acc_sh.at[di_v[b]], ssems[b]
                ).wait()
            ids.append(
                pltpu.async_copy(src_hbm.at[pl.ds(base + b * K, K)], si_v[b], isems[b])
            )
            dds.append(
                pltpu.async_copy(dst_hbm.at[pl.ds(base + b * K, K)], di_v[b], dsems[b])
            )
        gds = []
        for b in range(UNROLL):
            ids[b].wait()
            gds.append(
                pltpu.async_copy(h_hbm.at[si_v[b]], rows_v.at[b], gsems[b])
            )
        for b in range(UNROLL):
            gds[b].wait()
            dds[b].wait()
            pltpu.async_copy(rows_v.at[b], acc_sh.at[di_v[b]], ssems[b], add=True)

    body(0, False)

    def outer(t, carry):
        body(t, True)
        return carry

    lax.fori_loop(1, OUTER, outer, 0)
    for b in range(UNROLL):
        pltpu.make_async_copy(rows_v.at[b], acc_sh.at[di_v[b]], ssems[b]).wait()
    plsc.subcore_barrier()
    pltpu.sync_copy(
        acc_sh.at[pl.ds(s * ZPT, ZPT)], out_hbm.at[pl.ds(c * NPAD + s * ZPT, ZPT)]
    )


@functools.partial(
    pl.kernel,
    out_type=jax.ShapeDtypeStruct((NC * NPAD, D), jnp.float32),
    mesh=_mesh,
    scratch_types=[
        [pltpu.VMEM((K,), jnp.int32)] * UNROLL,
        pltpu.VMEM((K, D), jnp.float32),
        pltpu.VMEM_SHARED((NPAD, D), jnp.float32),
        [pltpu.SemaphoreType.DMA] * UNROLL,
        [pltpu.SemaphoreType.DMA] * UNROLL,
    ],
)
def _deg_partials(dst_hbm, out_hbm, di_v, ones_v, acc_sh, ssems, dsems):
    c = lax.axis_index("c")
    s = lax.axis_index("s")
    w = c * NS + s

    def zero(i, carry):
        for j in range(D // 16):
            ones_v[i, pl.ds(j * 16, 16)] = jnp.zeros((16,), jnp.float32)
        return carry

    lax.fori_loop(0, K, zero, 0)
    for j in range(ZPT // K):
        pltpu.sync_copy(ones_v, acc_sh.at[pl.ds(s * ZPT + j * K, K)])

    def fill(i, carry):
        for j in range(D // 16):
            ones_v[i, pl.ds(j * 16, 16)] = jnp.full((16,), 1.0, jnp.float32)
        return carry

    lax.fori_loop(0, K, fill, 0)
    plsc.subcore_barrier()

    def body(t, drain):
        base = w * EPW + t * UNROLL * K
        dds = []
        for b in range(UNROLL):
            if drain:
                pltpu.make_async_copy(ones_v, acc_sh.at[di_v[b]], ssems[b]).wait()
            dds.append(
                pltpu.async_copy(dst_hbm.at[pl.ds(base + b * K, K)], di_v[b], dsems[b])
            )
        for b in range(UNROLL):
            dds[b].wait()
            pltpu.async_copy(ones_v, acc_sh.at[di_v[b]], ssems[b], add=True)

    body(0, False)

    def outer(t, carry):
        body(t, True)
        return carry

    lax.fori_loop(1, OUTER, outer, 0)
    for b in range(UNROLL):
        pltpu.make_async_copy(ones_v, acc_sh.at[di_v[b]], ssems[b]).wait()
    plsc.subcore_barrier()
    pltpu.sync_copy(
        acc_sh.at[pl.ds(s * ZPT, ZPT)], out_hbm.at[pl.ds(c * NPAD + s * ZPT, ZPT)]
    )


def _agg(src, dst, h):
    return _agg_partials(src, dst, h).reshape(NC, NPAD, D)


BLK = 1024


def _dinv_block(degp):
    deg = 1.0 + degp[0] + degp[1]
    return lax.rsqrt(deg)


def _tc1_body(x_ref, w_ref, degp_ref, o_ref):
    dinv = _dinv_block(degp_ref[...])
    h = jnp.dot(x_ref[...], w_ref[...], preferred_element_type=jnp.float32)
    o_ref[...] = h * dinv


_tc1 = pl.pallas_call(
    _tc1_body,
    grid=(NPAD // BLK,),
    in_specs=[
        pl.BlockSpec((BLK, D), lambda i: (i, 0)),
        pl.BlockSpec((D, D), lambda i: (0, 0)),
        pl.BlockSpec((NC, BLK, 1), lambda i: (0, i, 0)),
    ],
    out_specs=pl.BlockSpec((BLK, D), lambda i: (i, 0)),
    out_shape=jax.ShapeDtypeStruct((NPAD, D), jnp.float32),
)


def _tc2_body(p_ref, hp_ref, degp_ref, b_ref, w_ref, o_ref):
    dinv = _dinv_block(degp_ref[...])
    ssum = p_ref[0] + p_ref[1] + hp_ref[...]
    a = jnp.maximum(ssum * dinv + b_ref[...], 0.0)
    h = jnp.dot(a, w_ref[...], preferred_element_type=jnp.float32)
    o_ref[...] = h * dinv


_tc2 = pl.pallas_call(
    _tc2_body,
    grid=(NPAD // BLK,),
    in_specs=[
        pl.BlockSpec((NC, BLK, D), lambda i: (0, i, 0)),
        pl.BlockSpec((BLK, D), lambda i: (i, 0)),
        pl.BlockSpec((NC, BLK, 1), lambda i: (0, i, 0)),
        pl.BlockSpec((1, D), lambda i: (0, 0)),
        pl.BlockSpec((D, D), lambda i: (0, 0)),
    ],
    out_specs=pl.BlockSpec((BLK, D), lambda i: (i, 0)),
    out_shape=jax.ShapeDtypeStruct((NPAD, D), jnp.float32),
)


def _tc3_body(p_ref, hp_ref, degp_ref, b_ref, wc_ref, bc_ref, o_ref):
    dinv = _dinv_block(degp_ref[...])
    ssum = p_ref[0] + p_ref[1] + hp_ref[...]
    a = jnp.maximum(ssum * dinv + b_ref[...], 0.0)
    logits = jnp.dot(a, wc_ref[...], preferred_element_type=jnp.float32) + bc_ref[...]
    m = jnp.max(logits, axis=1, keepdims=True)
    lse = jnp.log(jnp.sum(jnp.exp(logits - m), axis=1, keepdims=True))
    o_ref[...] = logits - m - lse


def _make_tc3(n_classes):
    return pl.pallas_call(
        _tc3_body,
        grid=(NPAD // BLK,),
        in_specs=[
            pl.BlockSpec((NC, BLK, D), lambda i: (0, i, 0)),
            pl.BlockSpec((BLK, D), lambda i: (i, 0)),
            pl.BlockSpec((NC, BLK, 1), lambda i: (0, i, 0)),
            pl.BlockSpec((1, D), lambda i: (0, 0)),
            pl.BlockSpec((D, n_classes), lambda i: (0, 0)),
            pl.BlockSpec((1, n_classes), lambda i: (0, 0)),
        ],
        out_specs=pl.BlockSpec((BLK, n_classes), lambda i: (i, 0)),
        out_shape=jax.ShapeDtypeStruct((NPAD, n_classes), jnp.float32),
    )


def kernel(x, edge_index, W1, b1, W2, b2, Wc, bc):
    n_classes = Wc.shape[1]
    n = x.shape[0]
    e = edge_index.shape[1]
    pad = EPAD - e
    pad_ids = jnp.arange(pad, dtype=jnp.int32)
    src = jnp.concatenate([edge_index[0].astype(jnp.int32), pad_ids % n])
    dst = jnp.concatenate(
        [edge_index[1].astype(jnp.int32), n + pad_ids % (NPAD - n)]
    )
    xp = jnp.concatenate([x, jnp.zeros((NPAD - n, D), x.dtype)])

    degp = _deg_partials(dst).reshape(NC, NPAD, D)[:, :, 0:1]
    h1p = _tc1(xp, W1, degp)
    p1 = _agg(src, dst, h1p)
    h2p = _tc2(p1, h1p, degp, b1.reshape(1, D), W2)
    p2 = _agg(src, dst, h2p)
    out = _make_tc3(n_classes)(
        p2, h2p, degp, b2.reshape(1, D), Wc, bc.reshape(1, n_classes)
    )
    return out[:n]


# speedup vs baseline: 24.1767x; 1.0019x over previous
import functools

import jax
import jax.numpy as jnp
from jax import lax
from jax.experimental import pallas as pl
from jax.experimental.pallas import tpu as pltpu
from jax.experimental.pallas import tpu_sc as plsc

N_NODES = 10000
NPAD = 10240
D = 128
NC, NS = 2, 16
NW = NC * NS
K = 80
EPW = 10240
EPAD = EPW * NW
CHUNKS = EPW // K
ZPT = NPAD // NS

UNROLL = 4
OUTER = CHUNKS // UNROLL

_mesh = plsc.VectorSubcoreMesh(core_axis_name="c", subcore_axis_name="s")


@functools.partial(
    pl.kernel,
    out_type=jax.ShapeDtypeStruct((NC * NPAD, D), jnp.float32),
    mesh=_mesh,
    scratch_types=[
        [pltpu.VMEM((K,), jnp.int32)] * UNROLL,
        [pltpu.VMEM((K,), jnp.int32)] * UNROLL,
        pltpu.VMEM((UNROLL, K, D), jnp.float32),
        pltpu.VMEM_SHARED((NPAD, D), jnp.float32),
        [pltpu.SemaphoreType.DMA] * UNROLL,
        [pltpu.SemaphoreType.DMA] * UNROLL,
        [pltpu.SemaphoreType.DMA] * UNROLL,
        [pltpu.SemaphoreType.DMA] * UNROLL,
    ],
)
def _agg_partials(src_hbm, dst_hbm, h_hbm, out_hbm, si_v, di_v, rows_v, acc_sh,
                  gsems, ssems, isems, dsems):
    c = lax.axis_index("c")
    s = lax.axis_index("s")
    w = c * NS + s

    def zero(i, carry):
        for j in range(D // 16):
            rows_v[0, i, pl.ds(j * 16, 16)] = jnp.zeros((16,), jnp.float32)
        return carry

    lax.fori_loop(0, K, zero, 0)
    for j in range(ZPT // K):
        pltpu.sync_copy(rows_v.at[0], acc_sh.at[pl.ds(s * ZPT + j * K, K)])
    plsc.subcore_barrier()

    def body(t, drain):
        base = w * EPW + t * UNROLL * K
        ids, dds = [], []
        for b in range(UNROLL):
            if drain:
                pltpu.make_async_copy(
                    rows_v.at[b], acc_sh.at[di_v[b]], ssems[b]
                ).wait()
            ids.append(
                pltpu.async_copy(src_hbm.at[pl.ds(base + b * K, K)], si_v[b], isems[b])
            )
            dds.append(
                pltpu.async_copy(dst_hbm.at[pl.ds(base + b * K, K)], di_v[b], dsems[b])
            )
        gds = []
        for b in range(UNROLL):
            ids[b].wait()
            gds.append(
                pltpu.async_copy(h_hbm.at[si_v[b]], rows_v.at[b], gsems[b])
            )
        for b in range(UNROLL):
            gds[b].wait()
            dds[b].wait()
            pltpu.async_copy(rows_v.at[b], acc_sh.at[di_v[b]], ssems[b], add=True)

    body(0, False)

    def outer(t, carry):
        body(t, True)
        return carry

    lax.fori_loop(1, OUTER, outer, 0)
    for b in range(UNROLL):
        pltpu.make_async_copy(rows_v.at[b], acc_sh.at[di_v[b]], ssems[b]).wait()
    plsc.subcore_barrier()
    pltpu.sync_copy(
        acc_sh.at[pl.ds(s * ZPT, ZPT)], out_hbm.at[pl.ds(c * NPAD + s * ZPT, ZPT)]
    )


@functools.partial(
    pl.kernel,
    out_type=jax.ShapeDtypeStruct((NC * NPAD, D), jnp.float32),
    mesh=_mesh,
    scratch_types=[
        [pltpu.VMEM((K,), jnp.int32)] * UNROLL,
        pltpu.VMEM((K, D), jnp.float32),
        pltpu.VMEM_SHARED((NPAD, D), jnp.float32),
        [pltpu.SemaphoreType.DMA] * UNROLL,
        [pltpu.SemaphoreType.DMA] * UNROLL,
    ],
)
def _deg_partials(dst_hbm, out_hbm, di_v, ones_v, acc_sh, ssems, dsems):
    c = lax.axis_index("c")
    s = lax.axis_index("s")
    w = c * NS + s

    def zero(i, carry):
        for j in range(D // 16):
            ones_v[i, pl.ds(j * 16, 16)] = jnp.zeros((16,), jnp.float32)
        return carry

    lax.fori_loop(0, K, zero, 0)
    for j in range(ZPT // K):
        pltpu.sync_copy(ones_v, acc_sh.at[pl.ds(s * ZPT + j * K, K)])

    def fill(i, carry):
        for j in range(D // 16):
            ones_v[i, pl.ds(j * 16, 16)] = jnp.full((16,), 1.0, jnp.float32)
        return carry

    lax.fori_loop(0, K, fill, 0)
    plsc.subcore_barrier()

    def body(t, drain):
        base = w * EPW + t * UNROLL * K
        dds = []
        for b in range(UNROLL):
            if drain:
                pltpu.make_async_copy(ones_v, acc_sh.at[di_v[b]], ssems[b]).wait()
            dds.append(
                pltpu.async_copy(dst_hbm.at[pl.ds(base + b * K, K)], di_v[b], dsems[b])
            )
        for b in range(UNROLL):
            dds[b].wait()
            pltpu.async_copy(ones_v, acc_sh.at[di_v[b]], ssems[b], add=True)

    body(0, False)

    def outer(t, carry):
        body(t, True)
        return carry

    lax.fori_loop(1, OUTER, outer, 0)
    for b in range(UNROLL):
        pltpu.make_async_copy(ones_v, acc_sh.at[di_v[b]], ssems[b]).wait()
    plsc.subcore_barrier()
    pltpu.sync_copy(
        acc_sh.at[pl.ds(s * ZPT, ZPT)], out_hbm.at[pl.ds(c * NPAD + s * ZPT, ZPT)]
    )


def _agg(src, dst, h):
    return _agg_partials(src, dst, h).reshape(NC, NPAD, D)


BLK = 1024


def _dinv_block(degp):
    deg = 1.0 + degp[0] + degp[1]
    return lax.rsqrt(deg)


def _tc0_body(x_ref, w_ref, o_ref):
    o_ref[...] = jnp.dot(x_ref[...], w_ref[...], preferred_element_type=jnp.float32)


_tc0 = pl.pallas_call(
    _tc0_body,
    grid=(NPAD // BLK,),
    in_specs=[
        pl.BlockSpec((BLK, D), lambda i: (i, 0)),
        pl.BlockSpec((D, D), lambda i: (0, 0)),
    ],
    out_specs=pl.BlockSpec((BLK, D), lambda i: (i, 0)),
    out_shape=jax.ShapeDtypeStruct((NPAD, D), jnp.float32),
)


def _tc1_body(h_ref, degp_ref, o_ref):
    o_ref[...] = h_ref[...] * _dinv_block(degp_ref[...])


_tc1 = pl.pallas_call(
    _tc1_body,
    grid=(NPAD // BLK,),
    in_specs=[
        pl.BlockSpec((BLK, D), lambda i: (i, 0)),
        pl.BlockSpec((NC, BLK, 1), lambda i: (0, i, 0)),
    ],
    out_specs=pl.BlockSpec((BLK, D), lambda i: (i, 0)),
    out_shape=jax.ShapeDtypeStruct((NPAD, D), jnp.float32),
)


def _tc2_body(p_ref, hp_ref, degp_ref, b_ref, w_ref, o_ref):
    dinv = _dinv_block(degp_ref[...])
    ssum = p_ref[0] + p_ref[1] + hp_ref[...]
    a = jnp.maximum(ssum * dinv + b_ref[...], 0.0)
    h = jnp.dot(a, w_ref[...], preferred_element_type=jnp.float32)
    o_ref[...] = h * dinv


_tc2 = pl.pallas_call(
    _tc2_body,
    grid=(NPAD // BLK,),
    in_specs=[
        pl.BlockSpec((NC, BLK, D), lambda i: (0, i, 0)),
        pl.BlockSpec((BLK, D), lambda i: (i, 0)),
        pl.BlockSpec((NC, BLK, 1), lambda i: (0, i, 0)),
        pl.BlockSpec((1, D), lambda i: (0, 0)),
        pl.BlockSpec((D, D), lambda i: (0, 0)),
    ],
    out_specs=pl.BlockSpec((BLK, D), lambda i: (i, 0)),
    out_shape=jax.ShapeDtypeStruct((NPAD, D), jnp.float32),
)


def _tc3_body(p_ref, hp_ref, degp_ref, b_ref, wc_ref, bc_ref, o_ref):
    dinv = _dinv_block(degp_ref[...])
    ssum = p_ref[0] + p_ref[1] + hp_ref[...]
    a = jnp.maximum(ssum * dinv + b_ref[...], 0.0)
    logits = jnp.dot(a, wc_ref[...], preferred_element_type=jnp.float32) + bc_ref[...]
    m = jnp.max(logits, axis=1, keepdims=True)
    lse = jnp.log(jnp.sum(jnp.exp(logits - m), axis=1, keepdims=True))
    o_ref[...] = logits - m - lse


def _make_tc3(n_classes):
    return pl.pallas_call(
        _tc3_body,
        grid=(NPAD // BLK,),
        in_specs=[
            pl.BlockSpec((NC, BLK, D), lambda i: (0, i, 0)),
            pl.BlockSpec((BLK, D), lambda i: (i, 0)),
            pl.BlockSpec((NC, BLK, 1), lambda i: (0, i, 0)),
            pl.BlockSpec((1, D), lambda i: (0, 0)),
            pl.BlockSpec((D, n_classes), lambda i: (0, 0)),
            pl.BlockSpec((1, n_classes), lambda i: (0, 0)),
        ],
        out_specs=pl.BlockSpec((BLK, n_classes), lambda i: (i, 0)),
        out_shape=jax.ShapeDtypeStruct((NPAD, n_classes), jnp.float32),
    )


def kernel(x, edge_index, W1, b1, W2, b2, Wc, bc):
    n_classes = Wc.shape[1]
    n = x.shape[0]
    e = edge_index.shape[1]
    pad = EPAD - e
    pad_ids = jnp.arange(pad, dtype=jnp.int32)
    src = jnp.concatenate([edge_index[0].astype(jnp.int32), pad_ids % n])
    dst = jnp.concatenate(
        [edge_index[1].astype(jnp.int32), n + pad_ids % (NPAD - n)]
    )
    xp = jnp.concatenate([x, jnp.zeros((NPAD - n, D), x.dtype)])

    h1 = _tc0(xp, W1)
    degp = _deg_partials(dst).reshape(NC, NPAD, D)[:, :, 0:1]
    h1p = _tc1(h1, degp)
    p1 = _agg(src, dst, h1p)
    h2p = _tc2(p1, h1p, degp, b1.reshape(1, D), W2)
    p2 = _agg(src, dst, h2p)
    out = _make_tc3(n_classes)(
        p2, h2p, degp, b2.reshape(1, D), Wc, bc.reshape(1, n_classes)
    )
    return out[:n]


# speedup vs baseline: 24.2660x; 1.0037x over previous
import functools

import jax
import jax.numpy as jnp
from jax import lax
from jax.experimental import pallas as pl
from jax.experimental.pallas import tpu as pltpu
from jax.experimental.pallas import tpu_sc as plsc

N_NODES = 10000
NPAD = 10240
D = 128
NC, NS = 2, 16
NW = NC * NS
K = 80
EPW = 10240
EPAD = EPW * NW
CHUNKS = EPW // K
ZPT = NPAD // NS

UNROLL = 4
OUTER = CHUNKS // UNROLL

_mesh = plsc.VectorSubcoreMesh(core_axis_name="c", subcore_axis_name="s")


@functools.partial(
    pl.kernel,
    out_type=jax.ShapeDtypeStruct((NC * NPAD, D), jnp.float32),
    mesh=_mesh,
    scratch_types=[
        [pltpu.VMEM((K,), jnp.int32)] * UNROLL,
        [pltpu.VMEM((K,), jnp.int32)] * UNROLL,
        pltpu.VMEM((UNROLL, K, D), jnp.float32),
        pltpu.VMEM_SHARED((NPAD, D), jnp.float32),
        [pltpu.SemaphoreType.DMA] * UNROLL,
        [pltpu.SemaphoreType.DMA] * UNROLL,
        [pltpu.SemaphoreType.DMA] * UNROLL,
        [pltpu.SemaphoreType.DMA] * UNROLL,
    ],
)
def _agg_partials(src_hbm, dst_hbm, h_hbm, out_hbm, si_v, di_v, rows_v, acc_sh,
                  gsems, ssems, isems, dsems):
    c = lax.axis_index("c")
    s = lax.axis_index("s")
    w = c * NS + s

    def zero(i, carry):
        for j in range(D // 16):
            rows_v[0, i, pl.ds(j * 16, 16)] = jnp.zeros((16,), jnp.float32)
        return carry

    lax.fori_loop(0, K, zero, 0)
    for j in range(ZPT // K):
        pltpu.sync_copy(rows_v.at[0], acc_sh.at[pl.ds(s * ZPT + j * K, K)])
    plsc.subcore_barrier()

    def body(t, drain):
        base = w * EPW + t * UNROLL * K
        ids, dds = [], []
        for b in range(UNROLL):
            if drain:
                pltpu.make_async_copy(
                    rows_v.at[b], acc_sh.at[di_v[b]], ssems[b]
                ).wait()
            ids.append(
                pltpu.async_copy(src_hbm.at[pl.ds(base + b * K, K)], si_v[b], isems[b])
            )
            dds.append(
                pltpu.async_copy(dst_hbm.at[pl.ds(base + b * K, K)], di_v[b], dsems[b])
            )
        gds = []
        for b in range(UNROLL):
            ids[b].wait()
            gds.append(
                pltpu.async_copy(h_hbm.at[si_v[b]], rows_v.at[b], gsems[b])
            )
        for b in range(UNROLL):
            gds[b].wait()
            dds[b].wait()
            pltpu.async_copy(rows_v.at[b], acc_sh.at[di_v[b]], ssems[b], add=True)

    body(0, False)

    def outer(t, carry):
        body(t, True)
        return carry

    lax.fori_loop(1, OUTER, outer, 0)
    for b in range(UNROLL):
        pltpu.make_async_copy(rows_v.at[b], acc_sh.at[di_v[b]], ssems[b]).wait()
    plsc.subcore_barrier()
    pltpu.sync_copy(
        acc_sh.at[pl.ds(s * ZPT, ZPT)], out_hbm.at[pl.ds(c * NPAD + s * ZPT, ZPT)]
    )


@functools.partial(
    pl.kernel,
    out_type=jax.ShapeDtypeStruct((NC * NPAD, D), jnp.float32),
    mesh=_mesh,
    scratch_types=[
        [pltpu.VMEM((K,), jnp.int32)] * UNROLL,
        pltpu.VMEM((K, D), jnp.float32),
        pltpu.VMEM_SHARED((NPAD, D), jnp.float32),
        [pltpu.SemaphoreType.DMA] * UNROLL,
        [pltpu.SemaphoreType.DMA] * UNROLL,
    ],
)
def _deg_partials(dst_hbm, out_hbm, di_v, ones_v, acc_sh, ssems, dsems):
    c = lax.axis_index("c")
    s = lax.axis_index("s")
    w = c * NS + s

    def zero(i, carry):
        for j in range(D // 16):
            ones_v[i, pl.ds(j * 16, 16)] = jnp.zeros((16,), jnp.float32)
        return carry

    lax.fori_loop(0, K, zero, 0)
    for j in range(ZPT // K):
        pltpu.sync_copy(ones_v, acc_sh.at[pl.ds(s * ZPT + j * K, K)])

    def fill(i, carry):
        for j in range(D // 16):
            ones_v[i, pl.ds(j * 16, 16)] = jnp.full((16,), 1.0, jnp.float32)
        return carry

    lax.fori_loop(0, K, fill, 0)
    plsc.subcore_barrier()

    def body(t, drain):
        base = w * EPW + t * UNROLL * K
        dds = []
        for b in range(UNROLL):
            if drain:
                pltpu.make_async_copy(ones_v, acc_sh.at[di_v[b]], ssems[b]).wait()
            dds.append(
                pltpu.async_copy(dst_hbm.at[pl.ds(base + b * K, K)], di_v[b], dsems[b])
            )
        for b in range(UNROLL):
            dds[b].wait()
            pltpu.async_copy(ones_v, acc_sh.at[di_v[b]], ssems[b], add=True)

    body(0, False)

    def outer(t, carry):
        body(t, True)
        return carry

    lax.fori_loop(1, OUTER, outer, 0)
    for b in range(UNROLL):
        pltpu.make_async_copy(ones_v, acc_sh.at[di_v[b]], ssems[b]).wait()
    plsc.subcore_barrier()
    pltpu.sync_copy(
        acc_sh.at[pl.ds(s * ZPT, ZPT)], out_hbm.at[pl.ds(c * NPAD + s * ZPT, ZPT)]
    )


def _agg(src, dst, h):
    return _agg_partials(src, dst, h).reshape(NC, NPAD, D)


BLK = 1024


def _dinv_block(degp):
    deg = 1.0 + degp[0] + degp[1]
    return lax.rsqrt(deg)


def _tc0_body(x_ref, w_ref, o_ref):
    o_ref[...] = jnp.dot(x_ref[...], w_ref[...], preferred_element_type=jnp.float32)


_tc0 = pl.pallas_call(
    _tc0_body,
    grid=(N_NODES // 1000,),
    in_specs=[
        pl.BlockSpec((1000, D), lambda i: (i, 0)),
        pl.BlockSpec((D, D), lambda i: (0, 0)),
    ],
    out_specs=pl.BlockSpec((1000, D), lambda i: (i, 0)),
    out_shape=jax.ShapeDtypeStruct((NPAD, D), jnp.float32),
)


def _tc1_body(h_ref, degp_ref, o_ref):
    o_ref[...] = h_ref[...] * _dinv_block(degp_ref[...])


_tc1 = pl.pallas_call(
    _tc1_body,
    grid=(NPAD // BLK,),
    in_specs=[
        pl.BlockSpec((BLK, D), lambda i: (i, 0)),
        pl.BlockSpec((NC, BLK, 1), lambda i: (0, i, 0)),
    ],
    out_specs=pl.BlockSpec((BLK, D), lambda i: (i, 0)),
    out_shape=jax.ShapeDtypeStruct((NPAD, D), jnp.float32),
)


def _tc2_body(p_ref, hp_ref, degp_ref, b_ref, w_ref, o_ref):
    dinv = _dinv_block(degp_ref[...])
    ssum = p_ref[0] + p_ref[1] + hp_ref[...]
    a = jnp.maximum(ssum * dinv + b_ref[...], 0.0)
    h = jnp.dot(a, w_ref[...], preferred_element_type=jnp.float32)
    o_ref[...] = h * dinv


_tc2 = pl.pallas_call(
    _tc2_body,
    grid=(NPAD // BLK,),
    in_specs=[
        pl.BlockSpec((NC, BLK, D), lambda i: (0, i, 0)),
        pl.BlockSpec((BLK, D), lambda i: (i, 0)),
        pl.BlockSpec((NC, BLK, 1), lambda i: (0, i, 0)),
        pl.BlockSpec((1, D), lambda i: (0, 0)),
        pl.BlockSpec((D, D), lambda i: (0, 0)),
    ],
    out_specs=pl.BlockSpec((BLK, D), lambda i: (i, 0)),
    out_shape=jax.ShapeDtypeStruct((NPAD, D), jnp.float32),
)


def _tc3_body(p_ref, hp_ref, degp_ref, b_ref, wc_ref, bc_ref, o_ref):
    dinv = _dinv_block(degp_ref[...])
    ssum = p_ref[0] + p_ref[1] + hp_ref[...]
    a = jnp.maximum(ssum * dinv + b_ref[...], 0.0)
    logits = jnp.dot(a, wc_ref[...], preferred_element_type=jnp.float32) + bc_ref[...]
    m = jnp.max(logits, axis=1, keepdims=True)
    lse = jnp.log(jnp.sum(jnp.exp(logits - m), axis=1, keepdims=True))
    o_ref[...] = logits - m - lse


def _make_tc3(n_classes):
    return pl.pallas_call(
        _tc3_body,
        grid=(N_NODES // 1000,),
        in_specs=[
            pl.BlockSpec((NC, 1000, D), lambda i: (0, i, 0)),
            pl.BlockSpec((1000, D), lambda i: (i, 0)),
            pl.BlockSpec((NC, 1000, 1), lambda i: (0, i, 0)),
            pl.BlockSpec((1, D), lambda i: (0, 0)),
            pl.BlockSpec((D, n_classes), lambda i: (0, 0)),
            pl.BlockSpec((1, n_classes), lambda i: (0, 0)),
        ],
        out_specs=pl.BlockSpec((1000, n_classes), lambda i: (i, 0)),
        out_shape=jax.ShapeDtypeStruct((N_NODES, n_classes), jnp.float32),
    )


def kernel(x, edge_index, W1, b1, W2, b2, Wc, bc):
    n_classes = Wc.shape[1]
    n = x.shape[0]
    e = edge_index.shape[1]
    pad = EPAD - e
    pad_ids = jnp.arange(pad, dtype=jnp.int32)
    src = jnp.concatenate([edge_index[0].astype(jnp.int32), pad_ids % n])
    dst = jnp.concatenate(
        [edge_index[1].astype(jnp.int32), n + pad_ids % (NPAD - n)]
    )
    h1 = _tc0(x, W1)
    degp = _deg_partials(dst).reshape(NC, NPAD, D)[:, :, 0:1]
    h1p = _tc1(h1, degp)
    p1 = _agg(src, dst, h1p)
    h2p = _tc2(p1, h1p, degp, b1.reshape(1, D), W2)
    p2 = _agg(src, dst, h2p)
    return _make_tc3(n_classes)(
        p2, h2p, degp, b2.reshape(1, D), Wc, bc.reshape(1, n_classes)
    )


# speedup vs baseline: 26.3570x; 1.0862x over previous
import functools

import jax
import jax.numpy as jnp
from jax import lax
from jax.experimental import pallas as pl
from jax.experimental.pallas import tpu as pltpu
from jax.experimental.pallas import tpu_sc as plsc

N_NODES = 10000
NPAD = 10240
D = 128
NC, NS = 2, 16
NW = NC * NS
K = 80
EPW = 10240
EPAD = EPW * NW
CHUNKS = EPW // K
ZPT = NPAD // NS

UNROLL = 4
OUTER = CHUNKS // UNROLL

_mesh = plsc.VectorSubcoreMesh(core_axis_name="c", subcore_axis_name="s")


@functools.partial(
    pl.kernel,
    out_type=jax.ShapeDtypeStruct((NC * NPAD, D), jnp.float32),
    mesh=_mesh,
    scratch_types=[
        [pltpu.VMEM((K,), jnp.int32)] * UNROLL,
        [pltpu.VMEM((K,), jnp.int32)] * UNROLL,
        pltpu.VMEM((UNROLL, K, D), jnp.float32),
        pltpu.VMEM_SHARED((NPAD, D), jnp.float32),
        [pltpu.SemaphoreType.DMA] * UNROLL,
        [pltpu.SemaphoreType.DMA] * UNROLL,
        [pltpu.SemaphoreType.DMA] * UNROLL,
        [pltpu.SemaphoreType.DMA] * UNROLL,
    ],
)
def _agg_partials(src_hbm, dst_hbm, h_hbm, out_hbm, si_v, di_v, rows_v, acc_sh,
                  gsems, ssems, isems, dsems):
    c = lax.axis_index("c")
    s = lax.axis_index("s")
    w = c * NS + s

    def zero(i, carry):
        for j in range(D // 16):
            rows_v[0, i, pl.ds(j * 16, 16)] = jnp.zeros((16,), jnp.float32)
        return carry

    lax.fori_loop(0, K, zero, 0)
    for j in range(ZPT // K):
        pltpu.sync_copy(rows_v.at[0], acc_sh.at[pl.ds(s * ZPT + j * K, K)])
    plsc.subcore_barrier()

    def body(t, drain):
        base = w * EPW + t * UNROLL * K
        ids, dds = [], []
        for b in range(UNROLL):
            if drain:
                pltpu.make_async_copy(
                    rows_v.at[b], acc_sh.at[di_v[b]], ssems[b]
                ).wait()
            ids.append(
                pltpu.async_copy(src_hbm.at[pl.ds(base + b * K, K)], si_v[b], isems[b])
            )
            dds.append(
                pltpu.async_copy(dst_hbm.at[pl.ds(base + b * K, K)], di_v[b], dsems[b])
            )
        gds = []
        for b in range(UNROLL):
            ids[b].wait()
            gds.append(
                pltpu.async_copy(h_hbm.at[si_v[b]], rows_v.at[b], gsems[b])
            )
        for b in range(UNROLL):
            gds[b].wait()
            dds[b].wait()
            pltpu.async_copy(rows_v.at[b], acc_sh.at[di_v[b]], ssems[b], add=True)

    body(0, False)

    def outer(t, carry):
        body(t, True)
        return carry

    lax.fori_loop(1, OUTER, outer, 0)
    for b in range(UNROLL):
        pltpu.make_async_copy(rows_v.at[b], acc_sh.at[di_v[b]], ssems[b]).wait()
    plsc.subcore_barrier()
    pltpu.sync_copy(
        acc_sh.at[pl.ds(s * ZPT, ZPT)], out_hbm.at[pl.ds(c * NPAD + s * ZPT, ZPT)]
    )


@functools.partial(
    pl.kernel,
    out_type=jax.ShapeDtypeStruct((NC * NPAD,), jnp.float32),
    mesh=_mesh,
    scratch_types=[
        [pltpu.VMEM((K,), jnp.int32)] * UNROLL,
        pltpu.VMEM((K,), jnp.float32),
        pltpu.VMEM_SHARED((NPAD,), jnp.float32),
        [pltpu.SemaphoreType.DMA] * UNROLL,
        [pltpu.SemaphoreType.DMA] * UNROLL,
    ],
)
def _deg_partials(dst_hbm, out_hbm, di_v, ones_v, acc_sh, ssems, dsems):
    c = lax.axis_index("c")
    s = lax.axis_index("s")
    w = c * NS + s

    for j in range(K // 16):
        ones_v[pl.ds(j * 16, 16)] = jnp.zeros((16,), jnp.float32)
    for j in range(ZPT // K):
        pltpu.sync_copy(ones_v, acc_sh.at[pl.ds(s * ZPT + j * K, K)])
    for j in range(K // 16):
        ones_v[pl.ds(j * 16, 16)] = jnp.full((16,), 1.0, jnp.float32)
    plsc.subcore_barrier()

    def body(t, drain):
        base = w * EPW + t * UNROLL * K
        dds = []
        for b in range(UNROLL):
            if drain:
                pltpu.make_async_copy(ones_v, acc_sh.at[di_v[b]], ssems[b]).wait()
            dds.append(
                pltpu.async_copy(dst_hbm.at[pl.ds(base + b * K, K)], di_v[b], dsems[b])
            )
        for b in range(UNROLL):
            dds[b].wait()
            pltpu.async_copy(ones_v, acc_sh.at[di_v[b]], ssems[b], add=True)

    body(0, False)

    def outer(t, carry):
        body(t, True)
        return carry

    lax.fori_loop(1, OUTER, outer, 0)
    for b in range(UNROLL):
        pltpu.make_async_copy(ones_v, acc_sh.at[di_v[b]], ssems[b]).wait()
    plsc.subcore_barrier()
    pltpu.sync_copy(
        acc_sh.at[pl.ds(s * ZPT, ZPT)], out_hbm.at[pl.ds(c * NPAD + s * ZPT, ZPT)]
    )


def _agg(src, dst, h):
    return _agg_partials(src, dst, h).reshape(NC, NPAD, D)


BLK = 1024


def _dinv_block(degp):
    deg = 1.0 + degp[0] + degp[1]
    return lax.rsqrt(deg)


def _tc0_body(x_ref, w_ref, o_ref):
    o_ref[...] = jnp.dot(x_ref[...], w_ref[...], preferred_element_type=jnp.float32)


_tc0 = pl.pallas_call(
    _tc0_body,
    grid=(N_NODES // 1000,),
    in_specs=[
        pl.BlockSpec((1000, D), lambda i: (i, 0)),
        pl.BlockSpec((D, D), lambda i: (0, 0)),
    ],
    out_specs=pl.BlockSpec((1000, D), lambda i: (i, 0)),
    out_shape=jax.ShapeDtypeStruct((NPAD, D), jnp.float32),
)


def _tc1_body(h_ref, degp_ref, o_ref):
    o_ref[...] = h_ref[...] * _dinv_block(degp_ref[...])


_tc1 = pl.pallas_call(
    _tc1_body,
    grid=(NPAD // BLK,),
    in_specs=[
        pl.BlockSpec((BLK, D), lambda i: (i, 0)),
        pl.BlockSpec((NC, BLK, 1), lambda i: (0, i, 0)),
    ],
    out_specs=pl.BlockSpec((BLK, D), lambda i: (i, 0)),
    out_shape=jax.ShapeDtypeStruct((NPAD, D), jnp.float32),
)


def _tc2_body(p_ref, hp_ref, degp_ref, b_ref, w_ref, o_ref):
    dinv = _dinv_block(degp_ref[...])
    ssum = p_ref[0] + p_ref[1] + hp_ref[...]
    a = jnp.maximum(ssum * dinv + b_ref[...], 0.0)
    h = jnp.dot(a, w_ref[...], preferred_element_type=jnp.float32)
    o_ref[...] = h * dinv


_tc2 = pl.pallas_call(
    _tc2_body,
    grid=(NPAD // BLK,),
    in_specs=[
        pl.BlockSpec((NC, BLK, D), lambda i: (0, i, 0)),
        pl.BlockSpec((BLK, D), lambda i: (i, 0)),
        pl.BlockSpec((NC, BLK, 1), lambda i: (0, i, 0)),
        pl.BlockSpec((1, D), lambda i: (0, 0)),
        pl.BlockSpec((D, D), lambda i: (0, 0)),
    ],
    out_specs=pl.BlockSpec((BLK, D), lambda i: (i, 0)),
    out_shape=jax.ShapeDtypeStruct((NPAD, D), jnp.float32),
)


def _tc3_body(p_ref, hp_ref, degp_ref, b_ref, wc_ref, bc_ref, o_ref):
    dinv = _dinv_block(degp_ref[...])
    ssum = p_ref[0] + p_ref[1] + hp_ref[...]
    a = jnp.maximum(ssum * dinv + b_ref[...], 0.0)
    logits = jnp.dot(a, wc_ref[...], preferred_element_type=jnp.float32) + bc_ref[...]
    m = jnp.max(logits, axis=1, keepdims=True)
    lse = jnp.log(jnp.sum(jnp.exp(logits - m), axis=1, keepdims=True))
    o_ref[...] = logits - m - lse


def _make_tc3(n_classes):
    return pl.pallas_call(
        _tc3_body,
        grid=(N_NODES // 1000,),
        in_specs=[
            pl.BlockSpec((NC, 1000, D), lambda i: (0, i, 0)),
            pl.BlockSpec((1000, D), lambda i: (i, 0)),
            pl.BlockSpec((NC, 1000, 1), lambda i: (0, i, 0)),
            pl.BlockSpec((1, D), lambda i: (0, 0)),
            pl.BlockSpec((D, n_classes), lambda i: (0, 0)),
            pl.BlockSpec((1, n_classes), lambda i: (0, 0)),
        ],
        out_specs=pl.BlockSpec((1000, n_classes), lambda i: (i, 0)),
        out_shape=jax.ShapeDtypeStruct((N_NODES, n_classes), jnp.float32),
    )


def kernel(x, edge_index, W1, b1, W2, b2, Wc, bc):
    n_classes = Wc.shape[1]
    n = x.shape[0]
    e = edge_index.shape[1]
    pad = EPAD - e
    pad_ids = jnp.arange(pad, dtype=jnp.int32)
    src = jnp.concatenate([edge_index[0].astype(jnp.int32), pad_ids % n])
    dst = jnp.concatenate(
        [edge_index[1].astype(jnp.int32), n + pad_ids % (NPAD - n)]
    )
    h1 = _tc0(x, W1)
    degp = _deg_partials(dst).reshape(NC, NPAD, 1)
    h1p = _tc1(h1, degp)
    p1 = _agg(src, dst, h1p)
    h2p = _tc2(p1, h1p, degp, b1.reshape(1, D), W2)
    p2 = _agg(src, dst, h2p)
    return _make_tc3(n_classes)(
        p2, h2p, degp, b2.reshape(1, D), Wc, bc.reshape(1, n_classes)
    )
